# baseline (speedup 1.0000x reference)
import jax
import jax.numpy as jnp
from jax.experimental import pallas as pl
from jax.experimental.pallas import tpu as pltpu

_N, _D, _H, _C = 4096, 512, 512, 64
_BY = 1024
_NY = _N // _BY
_BM = 256
_NB = _N // _BM
_BL = 2048
_NL = _N // _BL
_P0 = _NY
_P1 = _NY + _NB
_P1B = _P1 + _NL
_P2 = _P1B + _NL


def _mega(feat_ref, w0_ref, b0_ref, w1_ref, b1_ref, wc_ref, bc_ref, adj_ref,
          out_ref, adjb_scr, y0_scr, y1_scr, y2_scr, w1b_scr, wcb_scr):
    i = pl.program_id(0)

    @pl.when(i == 0)
    def _():
        w1b_scr[...] = w1_ref[...].astype(jnp.bfloat16)
        wcb_scr[...] = wc_ref[...].astype(jnp.bfloat16)

    @pl.when(i < _NY)
    def _():
        y0_scr[pl.ds(i * _BY, _BY), :] = jnp.dot(
            feat_ref[...].astype(jnp.bfloat16),
            w0_ref[...].astype(jnp.bfloat16),
            preferred_element_type=jnp.float32,
        ).astype(jnp.bfloat16)

    @pl.when((i >= _P0) & (i < _P1))
    def _():
        a = adj_ref[...].astype(jnp.bfloat16)
        h = jnp.dot(a, y0_scr[...], preferred_element_type=jnp.float32)
        y1_scr[pl.ds((i - _P0) * _BM, _BM), :] = jnp.maximum(
            h + b0_ref[...][None, :], 0.0
        ).astype(jnp.bfloat16)
        adjb_scr[pl.ds((i - _P0) * _BM, _BM), :] = a

    @pl.when((i >= _P1) & (i < _P1 + _NL))
    def _():
        r = (i - _P1) * _BL
        y1_scr[pl.ds(r, _BL), :] = jnp.dot(
            y1_scr[pl.ds(r, _BL), :], w1b_scr[...],
            preferred_element_type=jnp.float32,
        ).astype(jnp.bfloat16)

    @pl.when((i >= _P1B) & (i < _P2))
    def _():
        r = (i - _P1B) * _BL
        a = adjb_scr[pl.ds(r, _BL), :]
        h = jnp.dot(a, y1_scr[...], preferred_element_type=jnp.float32)
        h = jnp.maximum(h + b1_ref[...][None, :], 0.0).astype(jnp.bfloat16)
        y2_scr[pl.ds(r, _BL), :] = jnp.dot(
            h, wcb_scr[...], preferred_element_type=jnp.float32
        ).astype(jnp.bfloat16)

    @pl.when(i >= _P2)
    def _():
        r = (i - _P2) * _BL
        a = adjb_scr[pl.ds(r, _BL), :]
        out_ref[...] = (
            jnp.dot(a, y2_scr[...], preferred_element_type=jnp.float32)
            + bc_ref[...][None, :]
        )


def kernel(features, adj, W0, b0, W1, b1, Wc, bc):
    full = lambda shape: pl.BlockSpec(shape, lambda i: (0, 0))

    out = pl.pallas_call(
        _mega,
        grid=(_P2 + _NL,),
        in_specs=[
            pl.BlockSpec((_BY, _D), lambda i: (jnp.minimum(i, _NY - 1), 0)),
            full((_D, _H)),
            pl.BlockSpec((_H,), lambda i: (0,)),
            full((_H, _H)),
            pl.BlockSpec((_H,), lambda i: (0,)),
            full((_H, _C)),
            pl.BlockSpec((_C,), lambda i: (0,)),
            pl.BlockSpec(
                (_BM, _N),
                lambda i: (jnp.clip(i - _P0, 0, _NB - 1), 0),
            ),
        ],
        out_specs=pl.BlockSpec(
            (_BL, _C), lambda i: (jnp.maximum(i - _P2, 0), 0)
        ),
        out_shape=jax.ShapeDtypeStruct((_N, _C), jnp.float32),
        scratch_shapes=[
            pltpu.VMEM((_N, _N), jnp.bfloat16),
            pltpu.VMEM((_N, _H), jnp.bfloat16),
            pltpu.VMEM((_N, _H), jnp.bfloat16),
            pltpu.VMEM((_N, _C), jnp.bfloat16),
            pltpu.VMEM((_H, _H), jnp.bfloat16),
            pltpu.VMEM((_H, _C), jnp.bfloat16),
        ],
        compiler_params=pltpu.CompilerParams(
            dimension_semantics=("arbitrary",),
            vmem_limit_bytes=64 * 1024 * 1024,
        ),
    )(features, W0, b0, W1, b1, Wc, bc, adj)

    return out


# speedup vs baseline: 1.0210x; 1.0210x over previous
import jax
import jax.numpy as jnp
from jax.experimental import pallas as pl
from jax.experimental.pallas import tpu as pltpu

_N, _D, _H, _C = 4096, 512, 512, 64
_BY = 1024
_NY = _N // _BY
_BM = 256
_NB = _N // _BM
_BL = 2048
_NL = _N // _BL
_P0 = _NY
_P1 = _NY + _NB
_P2 = _P1 + _NL


def _mega(feat_ref, w0_ref, b0_ref, w1_ref, b1_ref, wc_ref, bc_ref, adj_ref,
          out_ref, adjb_scr, y0_scr, y1_scr, y2_scr, w1b_scr, wcb_scr):
    i = pl.program_id(0)

    @pl.when(i == 0)
    def _():
        w1b_scr[...] = w1_ref[...].astype(jnp.bfloat16)
        wcb_scr[...] = wc_ref[...].astype(jnp.bfloat16)

    @pl.when(i < _NY)
    def _():
        y0_scr[pl.ds(i * _BY, _BY), :] = jnp.dot(
            feat_ref[...].astype(jnp.bfloat16),
            w0_ref[...].astype(jnp.bfloat16),
            preferred_element_type=jnp.float32,
        ).astype(jnp.bfloat16)

    @pl.when((i >= _P0) & (i < _P1))
    def _():
        a = adj_ref[...].astype(jnp.bfloat16)
        h = jnp.dot(a, y0_scr[...], preferred_element_type=jnp.float32)
        h = jnp.maximum(h + b0_ref[...][None, :], 0.0).astype(jnp.bfloat16)
        y1_scr[pl.ds((i - _P0) * _BM, _BM), :] = jnp.dot(
            h, w1b_scr[...], preferred_element_type=jnp.float32
        ).astype(jnp.bfloat16)
        adjb_scr[pl.ds((i - _P0) * _BM, _BM), :] = a

    @pl.when((i >= _P1) & (i < _P2))
    def _():
        r = (i - _P1) * _BL
        a = adjb_scr[pl.ds(r, _BL), :]
        h = jnp.dot(a, y1_scr[...], preferred_element_type=jnp.float32)
        h = jnp.maximum(h + b1_ref[...][None, :], 0.0).astype(jnp.bfloat16)
        y2_scr[pl.ds(r, _BL), :] = jnp.dot(
            h, wcb_scr[...], preferred_element_type=jnp.float32
        ).astype(jnp.bfloat16)

    @pl.when(i >= _P2)
    def _():
        r = (i - _P2) * _BL
        a = adjb_scr[pl.ds(r, _BL), :]
        out_ref[...] = (
            jnp.dot(a, y2_scr[...], preferred_element_type=jnp.float32)
            + bc_ref[...][None, :]
        )


def kernel(features, adj, W0, b0, W1, b1, Wc, bc):
    full = lambda shape: pl.BlockSpec(shape, lambda i: (0, 0))

    out = pl.pallas_call(
        _mega,
        grid=(_P2 + _NL,),
        in_specs=[
            pl.BlockSpec((_BY, _D), lambda i: (jnp.minimum(i, _NY - 1), 0)),
            full((_D, _H)),
            pl.BlockSpec((_H,), lambda i: (0,)),
            full((_H, _H)),
            pl.BlockSpec((_H,), lambda i: (0,)),
            full((_H, _C)),
            pl.BlockSpec((_C,), lambda i: (0,)),
            pl.BlockSpec(
                (_BM, _N),
                lambda i: (jnp.clip(i - _P0, 0, _NB - 1), 0),
            ),
        ],
        out_specs=pl.BlockSpec(
            (_BL, _C), lambda i: (jnp.maximum(i - _P2, 0), 0)
        ),
        out_shape=jax.ShapeDtypeStruct((_N, _C), jnp.float32),
        scratch_shapes=[
            pltpu.VMEM((_N, _N), jnp.bfloat16),
            pltpu.VMEM((_N, _H), jnp.bfloat16),
            pltpu.VMEM((_N, _H), jnp.bfloat16),
            pltpu.VMEM((_N, _C), jnp.bfloat16),
            pltpu.VMEM((_H, _H), jnp.bfloat16),
            pltpu.VMEM((_H, _C), jnp.bfloat16),
        ],
        compiler_params=pltpu.CompilerParams(
            dimension_semantics=("arbitrary",),
            vmem_limit_bytes=64 * 1024 * 1024,
        ),
    )(features, W0, b0, W1, b1, Wc, bc, adj)

    return out
